# baseline (speedup 1.0000x reference)
import numpy as np

import jax
import jax.numpy as jnp
from jax import lax
from jax.experimental import pallas as pl
from jax.experimental.pallas import tpu as pltpu
from jax.experimental.pallas import tpu_sc as plsc

_B, _C, _H, _W = 64, 384, 28, 28
_HW = _H * _W
_N = _C * _HW
_K = int(0.1 * _N)
_TARGET = float(np.float32(_K / _N))

_NW = 32
_ROWS_PER_W = _B // _NW
_NCHUNK = 16
_SLABS = _HW // _NCHUNK
_CVREG = _C // 16
_NBIN = 4096
_NGRP = _NBIN // 16
_BANDCAP = 1024


def _key_of(bv):
    b = plsc.bitcast(bv, jnp.int32)
    return b ^ ((b >> jnp.int32(31)) & jnp.int32(0x7FFFFFFF))


def _sc_body(x_hbm, duty_hbm, out_hbm, xb0, xb1, hist, band, totals, bfl,
             dbuf, gsum, isem0, isem1, osem0, osem1):
    lanes = lax.iota(jnp.int32, 16)
    wid = lax.axis_index("s") * 2 + lax.axis_index("c")

    pltpu.sync_copy(duty_hbm, dbuf)
    def _bf_step(i, _):
        v = dbuf[pl.ds(i * 16, 16)]
        bfl[pl.ds(i * 16, 16)] = jnp.exp(jnp.float32(_TARGET) - v)
        return 0
    lax.fori_loop(0, _C // 16, _bf_step, 0)

    def _row_body(r, _):
        row = wid * _ROWS_PER_W + r

        def src(c):
            return x_hbm.at[pl.ds(c * _SLABS, _SLABS), pl.ds(row, 1), :]

        def dst(c):
            return out_hbm.at[pl.ds(c * _SLABS, _SLABS), pl.ds(row, 1), :]

        @plsc.parallel_loop(0, _NBIN, unroll=8)
        def _zero(i):
            hist[pl.ds(i * 16, 16)] = jnp.zeros((16,), jnp.int32)

        def _p1_process(buf):
            def _p1_cv(cv, _):
                bfv = bfl[pl.ds(cv * 16, 16)]
                @plsc.parallel_loop(0, _SLABS, unroll=7)
                def _p1_s(s):
                    xv = buf[s, 0, pl.ds(cv * 16, 16)]
                    key = _key_of(xv * bfv)
                    binv = (key >> jnp.int32(20)) + jnp.int32(2048)
                    sw = (binv + lanes) & jnp.int32(_NBIN - 1)
                    idx = lanes * jnp.int32(_NBIN) + sw
                    plsc.addupdate_scatter(hist, [idx],
                                           jnp.full((16,), 1, jnp.int32))
                return 0
            lax.fori_loop(0, _CVREG, _p1_cv, 0)

        pltpu.async_copy(src(0), xb0, isem0)
        def _p1_pair(cc, _):
            c0 = cc * 2
            pltpu.async_copy(src(c0 + 1), xb1, isem1)
            pltpu.make_async_copy(src(c0), xb0, isem0).wait()
            _p1_process(xb0)
            @pl.when(cc < _NCHUNK // 2 - 1)
            def _():
                pltpu.async_copy(src(c0 + 2), xb0, isem0)
            pltpu.make_async_copy(src(c0 + 1), xb1, isem1).wait()
            _p1_process(xb1)
            return 0
        lax.fori_loop(0, _NCHUNK // 2, _p1_pair, 0)

        pltpu.async_copy(src(0), xb0, isem0)

        @plsc.parallel_loop(0, _NGRP, unroll=2)
        def _tot(g):
            def _tl(l, acc):
                sw = (g * 16 + lanes + l) & jnp.int32(_NBIN - 1)
                return acc + plsc.load_gather(hist, [l * jnp.int32(_NBIN) + sw])
            acc = lax.fori_loop(0, 16, _tl, jnp.zeros((16,), jnp.int32),
                                unroll=16)
            totals[pl.ds(g * 16, 16)] = acc
            gsum[g] = jnp.sum(acc)

        def _scan(i, carry):
            found, cum, gstar, cumstar = carry
            g = jnp.int32(_NGRP - 1) - i
            s = gsum[g]
            crosses = jnp.logical_and(jnp.logical_not(found), cum + s >= _K)
            gstar = jnp.where(crosses, g, gstar)
            cumstar = jnp.where(crosses, cum, cumstar)
            cum = jnp.where(found | crosses, cum, cum + s)
            found = found | crosses
            return found, cum, gstar, cumstar
        _, _, gstar, cumstar = lax.fori_loop(
            0, _NGRP, _scan,
            (jnp.bool_(False), jnp.int32(0), jnp.int32(0), jnp.int32(0)))

        t = totals[pl.ds(gstar * 16, 16)]
        rv = lax.rev(t, (0,))
        suff = lax.rev(plsc.cumsum(rv), (0,))
        cond = (cumstar + suff) >= _K
        npos = plsc.all_reduce_population_count(cond)
        b_loc = jnp.max(npos) - 1
        b_star = gstar * 16 + b_loc
        mask2 = lanes > b_loc
        cnt_gt = cumstar + jnp.sum(jnp.where(mask2, t, jnp.int32(0)))
        bin_hi = b_star - jnp.int32(2048)

        def _p3_process(buf, off_l):
            def _p3_cv(cv, off_l):
                bfv = bfl[pl.ds(cv * 16, 16)]
                def _p3_s(s, off_l):
                    xv = buf[s, 0, pl.ds(cv * 16, 16)]
                    key = _key_of(xv * bfv)
                    inband = (key >> jnp.int32(20)) == bin_hi
                    guard = jnp.logical_and(inband,
                                            off_l < jnp.int32(_BANDCAP))
                    idx = lanes * jnp.int32(_BANDCAP) + off_l
                    plsc.store_scatter(band, [idx], key, mask=guard)
                    return off_l + jnp.where(guard, jnp.int32(1),
                                             jnp.int32(0))
                return plsc.parallel_loop(0, _SLABS, unroll=7,
                                          carry=off_l)(_p3_s)
            return lax.fori_loop(0, _CVREG, _p3_cv, off_l)

        def _p3_pair(cc, off_l):
            c0 = cc * 2
            pltpu.async_copy(src(c0 + 1), xb1, isem1)
            pltpu.make_async_copy(src(c0), xb0, isem0).wait()
            off_l = _p3_process(xb0, off_l)
            @pl.when(cc < _NCHUNK // 2 - 1)
            def _():
                pltpu.async_copy(src(c0 + 2), xb0, isem0)
            pltpu.make_async_copy(src(c0 + 1), xb1, isem1).wait()
            off_l = _p3_process(xb1, off_l)
            return off_l
        off_l = lax.fori_loop(0, _NCHUNK // 2, _p3_pair,
                              jnp.zeros((16,), jnp.int32))
        mj = jnp.max(off_l)

        pltpu.async_copy(src(0), xb0, isem0)
        pltpu.async_copy(src(1), xb1, isem1)

        key_lo = jnp.left_shift(bin_hi, jnp.int32(20))
        def _bit(i, t):
            bit = jnp.int32(19) - i
            cand = t + jnp.left_shift(jnp.int32(1), bit)
            def _cnt(j, acc):
                kv = plsc.load_gather(band, [lanes * jnp.int32(_BANDCAP) + j])
                valid = jnp.logical_and(j < off_l, kv >= cand)
                return acc + jnp.where(valid, jnp.int32(1), jnp.int32(0))
            acc = lax.fori_loop(0, mj, _cnt, jnp.zeros((16,), jnp.int32))
            total = cnt_gt + jnp.sum(acc)
            return jnp.where(total >= _K, cand, t)
        thr = lax.fori_loop(0, 20, _bit, key_lo)

        def _p5_process(buf):
            def _p5_cv(cv, _):
                bfv = bfl[pl.ds(cv * 16, 16)]
                @plsc.parallel_loop(0, _SLABS, unroll=7)
                def _p5_s(s):
                    xv = buf[s, 0, pl.ds(cv * 16, 16)]
                    key = _key_of(xv * bfv)
                    buf[s, 0, pl.ds(cv * 16, 16)] = jnp.where(
                        key >= thr, xv, jnp.float32(0.0))
                return 0
            lax.fori_loop(0, _CVREG, _p5_cv, 0)

        def _p5_pair(cc, _):
            c0 = cc * 2
            pltpu.make_async_copy(src(c0), xb0, isem0).wait()
            _p5_process(xb0)
            pltpu.async_copy(xb0, dst(c0), osem0)
            pltpu.make_async_copy(src(c0 + 1), xb1, isem1).wait()
            _p5_process(xb1)
            pltpu.async_copy(xb1, dst(c0 + 1), osem1)
            @pl.when(cc < _NCHUNK // 2 - 1)
            def _():
                pltpu.make_async_copy(xb0, dst(c0), osem0).wait()
                pltpu.async_copy(src(c0 + 2), xb0, isem0)
                pltpu.make_async_copy(xb1, dst(c0 + 1), osem1).wait()
                pltpu.async_copy(src(c0 + 3), xb1, isem1)
            return 0
        lax.fori_loop(0, _NCHUNK // 2, _p5_pair, 0)
        pltpu.make_async_copy(xb0, dst(_NCHUNK - 2), osem0).wait()
        pltpu.make_async_copy(xb1, dst(_NCHUNK - 1), osem1).wait()
        return 0

    lax.fori_loop(0, _ROWS_PER_W, _row_body, 0)


@jax.jit
def kernel(x, duty_cycles):
    xt = jnp.transpose(x, (2, 3, 0, 1)).reshape(_HW, _B, _C)
    duty = duty_cycles.reshape(_C)
    mesh = plsc.VectorSubcoreMesh(
        core_axis_name="c", subcore_axis_name="s", num_cores=2,
        num_subcores=16)
    out = pl.kernel(
        _sc_body,
        out_type=jax.ShapeDtypeStruct((_HW, _B, _C), jnp.float32),
        mesh=mesh,
        compiler_params=pltpu.CompilerParams(needs_layout_passes=False),
        scratch_types=[
            pltpu.VMEM((_SLABS, 1, _C), jnp.float32),
            pltpu.VMEM((_SLABS, 1, _C), jnp.float32),
            pltpu.VMEM((_NBIN * 16,), jnp.int32),
            pltpu.VMEM((_BANDCAP * 16,), jnp.int32),
            pltpu.VMEM((_NBIN,), jnp.int32),
            pltpu.VMEM((_C,), jnp.float32),
            pltpu.VMEM((_C,), jnp.float32),
            pltpu.SMEM((_NGRP,), jnp.int32),
            pltpu.SemaphoreType.DMA,
            pltpu.SemaphoreType.DMA,
            pltpu.SemaphoreType.DMA,
            pltpu.SemaphoreType.DMA,
        ],
    )(xt, duty)
    return jnp.transpose(out.reshape(_H, _W, _B, _C), (2, 3, 0, 1))


# speedup vs baseline: 1.3181x; 1.3181x over previous
import numpy as np

import jax
import jax.numpy as jnp
from jax import lax
from jax.experimental import pallas as pl
from jax.experimental.pallas import tpu as pltpu
from jax.experimental.pallas import tpu_sc as plsc

_B, _C, _H, _W = 64, 384, 28, 28
_HW = _H * _W
_N = _C * _HW
_K = int(0.1 * _N)
_TARGET = float(np.float32(_K / _N))

_NW = 32
_ROWS_PER_W = _B // _NW
_NCHUNK = 16
_SLABS = _HW // _NCHUNK
_CVREG = _C // 16
_NBIN = 4096
_NGRP = _NBIN // 16
_BANDCAP = 1024


def _key_of(bv):
    b = plsc.bitcast(bv, jnp.int32)
    return b ^ ((b >> jnp.int32(31)) & jnp.int32(0x7FFFFFFF))


def _sc_body(x_hbm, duty_hbm, out_hbm, xb0, xb1, hist, band, totals, bfl,
             dbuf, gsum, isem0, isem1, osem0, osem1):
    lanes = lax.iota(jnp.int32, 16)
    wid = lax.axis_index("s") * 2 + lax.axis_index("c")

    pltpu.sync_copy(duty_hbm, dbuf)
    def _bf_step(i, _):
        v = dbuf[pl.ds(i * 16, 16)]
        bfl[pl.ds(i * 16, 16)] = jnp.exp(jnp.float32(_TARGET) - v)
        return 0
    lax.fori_loop(0, _C // 16, _bf_step, 0)

    def _row_body(r, _):
        row = wid * _ROWS_PER_W + r

        def src(c):
            return x_hbm.at[pl.ds(c * _SLABS, _SLABS), pl.ds(row, 1), :]

        def dst(c):
            return out_hbm.at[pl.ds(c * _SLABS, _SLABS), pl.ds(row, 1), :]

        @plsc.parallel_loop(0, _NBIN, unroll=8)
        def _zero(i):
            hist[pl.ds(i * 16, 16)] = jnp.zeros((16,), jnp.int32)

        def _p1_process(buf):
            def _p1_cv(cv, _):
                bfv = bfl[pl.ds(cv * 16, 16)]
                @plsc.parallel_loop(0, _SLABS, unroll=7)
                def _p1_s(s):
                    xv = buf[s, 0, pl.ds(cv * 16, 16)]
                    key = _key_of(xv * bfv)
                    binv = (key >> jnp.int32(20)) + jnp.int32(2048)
                    idx = lanes * jnp.int32(_NBIN) + binv
                    plsc.addupdate_scatter(hist, [idx],
                                           jnp.full((16,), 1, jnp.int32))
                return 0
            lax.fori_loop(0, _CVREG, _p1_cv, 0)

        pltpu.async_copy(src(0), xb0, isem0)
        def _p1_pair(cc, _):
            c0 = cc * 2
            pltpu.async_copy(src(c0 + 1), xb1, isem1)
            pltpu.make_async_copy(src(c0), xb0, isem0).wait()
            _p1_process(xb0)
            @pl.when(cc < _NCHUNK // 2 - 1)
            def _():
                pltpu.async_copy(src(c0 + 2), xb0, isem0)
            pltpu.make_async_copy(src(c0 + 1), xb1, isem1).wait()
            _p1_process(xb1)
            return 0
        lax.fori_loop(0, _NCHUNK // 2, _p1_pair, 0)

        pltpu.async_copy(src(0), xb0, isem0)

        @plsc.parallel_loop(0, _NGRP, unroll=2)
        def _tot(g):
            def _tl(l, acc):
                return acc + hist[pl.ds(l * _NBIN + g * 16, 16)]
            acc = lax.fori_loop(0, 16, _tl, jnp.zeros((16,), jnp.int32),
                                unroll=16)
            totals[pl.ds(g * 16, 16)] = acc
            gsum[g] = jnp.sum(acc)

        def _scan(i, carry):
            found, cum, gstar, cumstar = carry
            g = jnp.int32(_NGRP - 1) - i
            s = gsum[g]
            crosses = jnp.logical_and(jnp.logical_not(found), cum + s >= _K)
            gstar = jnp.where(crosses, g, gstar)
            cumstar = jnp.where(crosses, cum, cumstar)
            cum = jnp.where(found | crosses, cum, cum + s)
            found = found | crosses
            return found, cum, gstar, cumstar
        _, _, gstar, cumstar = lax.fori_loop(
            0, _NGRP, _scan,
            (jnp.bool_(False), jnp.int32(0), jnp.int32(0), jnp.int32(0)))

        t = totals[pl.ds(gstar * 16, 16)]
        rv = lax.rev(t, (0,))
        suff = lax.rev(plsc.cumsum(rv), (0,))
        cond = (cumstar + suff) >= _K
        npos = plsc.all_reduce_population_count(cond)
        b_loc = jnp.max(npos) - 1
        b_star = gstar * 16 + b_loc
        mask2 = lanes > b_loc
        cnt_gt = cumstar + jnp.sum(jnp.where(mask2, t, jnp.int32(0)))
        bin_hi = b_star - jnp.int32(2048)

        def _p3_process(buf, off_l):
            def _p3_cv(cv, off_l):
                bfv = bfl[pl.ds(cv * 16, 16)]
                def _p3_s(s, off_l):
                    xv = buf[s, 0, pl.ds(cv * 16, 16)]
                    key = _key_of(xv * bfv)
                    inband = (key >> jnp.int32(20)) == bin_hi
                    guard = jnp.logical_and(inband,
                                            off_l < jnp.int32(_BANDCAP))
                    idx = off_l * jnp.int32(16) + lanes
                    plsc.store_scatter(band, [idx], key, mask=guard)
                    return off_l + jnp.where(guard, jnp.int32(1),
                                             jnp.int32(0))
                return plsc.parallel_loop(0, _SLABS, unroll=7,
                                          carry=off_l)(_p3_s)
            return lax.fori_loop(0, _CVREG, _p3_cv, off_l)

        def _p3_pair(cc, off_l):
            c0 = cc * 2
            pltpu.async_copy(src(c0 + 1), xb1, isem1)
            pltpu.make_async_copy(src(c0), xb0, isem0).wait()
            off_l = _p3_process(xb0, off_l)
            @pl.when(cc < _NCHUNK // 2 - 1)
            def _():
                pltpu.async_copy(src(c0 + 2), xb0, isem0)
            pltpu.make_async_copy(src(c0 + 1), xb1, isem1).wait()
            off_l = _p3_process(xb1, off_l)
            return off_l
        off_l = lax.fori_loop(0, _NCHUNK // 2, _p3_pair,
                              jnp.zeros((16,), jnp.int32))
        mj = jnp.max(off_l)

        pltpu.async_copy(src(0), xb0, isem0)
        pltpu.async_copy(src(1), xb1, isem1)

        key_lo = jnp.left_shift(bin_hi, jnp.int32(20))
        def _bit(i, t):
            bit = jnp.int32(19) - i
            cand = t + jnp.left_shift(jnp.int32(1), bit)
            def _cnt(j, acc):
                kv = band[pl.ds(j * 16, 16)]
                valid = jnp.logical_and(j < off_l, kv >= cand)
                return acc + jnp.where(valid, jnp.int32(1), jnp.int32(0))
            acc = lax.fori_loop(0, mj, _cnt, jnp.zeros((16,), jnp.int32))
            total = cnt_gt + jnp.sum(acc)
            return jnp.where(total >= _K, cand, t)
        thr = lax.fori_loop(0, 20, _bit, key_lo)

        def _p5_process(buf):
            def _p5_cv(cv, _):
                bfv = bfl[pl.ds(cv * 16, 16)]
                @plsc.parallel_loop(0, _SLABS, unroll=7)
                def _p5_s(s):
                    xv = buf[s, 0, pl.ds(cv * 16, 16)]
                    key = _key_of(xv * bfv)
                    buf[s, 0, pl.ds(cv * 16, 16)] = jnp.where(
                        key >= thr, xv, jnp.float32(0.0))
                return 0
            lax.fori_loop(0, _CVREG, _p5_cv, 0)

        def _p5_pair(cc, _):
            c0 = cc * 2
            pltpu.make_async_copy(src(c0), xb0, isem0).wait()
            _p5_process(xb0)
            pltpu.async_copy(xb0, dst(c0), osem0)
            pltpu.make_async_copy(src(c0 + 1), xb1, isem1).wait()
            _p5_process(xb1)
            pltpu.async_copy(xb1, dst(c0 + 1), osem1)
            @pl.when(cc < _NCHUNK // 2 - 1)
            def _():
                pltpu.make_async_copy(xb0, dst(c0), osem0).wait()
                pltpu.async_copy(src(c0 + 2), xb0, isem0)
                pltpu.make_async_copy(xb1, dst(c0 + 1), osem1).wait()
                pltpu.async_copy(src(c0 + 3), xb1, isem1)
            return 0
        lax.fori_loop(0, _NCHUNK // 2, _p5_pair, 0)
        pltpu.make_async_copy(xb0, dst(_NCHUNK - 2), osem0).wait()
        pltpu.make_async_copy(xb1, dst(_NCHUNK - 1), osem1).wait()
        return 0

    lax.fori_loop(0, _ROWS_PER_W, _row_body, 0)


@jax.jit
def kernel(x, duty_cycles):
    xt = jnp.transpose(x, (2, 3, 0, 1)).reshape(_HW, _B, _C)
    duty = duty_cycles.reshape(_C)
    mesh = plsc.VectorSubcoreMesh(
        core_axis_name="c", subcore_axis_name="s", num_cores=2,
        num_subcores=16)
    out = pl.kernel(
        _sc_body,
        out_type=jax.ShapeDtypeStruct((_HW, _B, _C), jnp.float32),
        mesh=mesh,
        compiler_params=pltpu.CompilerParams(needs_layout_passes=False),
        scratch_types=[
            pltpu.VMEM((_SLABS, 1, _C), jnp.float32),
            pltpu.VMEM((_SLABS, 1, _C), jnp.float32),
            pltpu.VMEM((_NBIN * 16,), jnp.int32),
            pltpu.VMEM((_BANDCAP * 16,), jnp.int32),
            pltpu.VMEM((_NBIN,), jnp.int32),
            pltpu.VMEM((_C,), jnp.float32),
            pltpu.VMEM((_C,), jnp.float32),
            pltpu.SMEM((_NGRP,), jnp.int32),
            pltpu.SemaphoreType.DMA,
            pltpu.SemaphoreType.DMA,
            pltpu.SemaphoreType.DMA,
            pltpu.SemaphoreType.DMA,
        ],
    )(xt, duty)
    return jnp.transpose(out.reshape(_H, _W, _B, _C), (2, 3, 0, 1))
